# baseline (speedup 1.0000x reference)
import functools

import jax
import jax.numpy as jnp
from jax import lax
from jax.experimental import pallas as pl
from jax.experimental.pallas import tpu as pltpu
from jax.experimental.pallas import tpu_sc as plsc

N_NODES = 10000
D = 128
N_EDGES = 320000

NC = 2
NS = 16
NW = NC * NS
K = 80
NCHUNK = 128
NBLK = 4
BC = NCHUNK // NBLK
BPAIRS = BC // 2
EPW = NCHUNK * K
E_PAD = NW * EPW
ROWS_PER_TILE = 632
N_PAD = ROWS_PER_TILE * NS


def _mm_body(x_ref, w_ref, o_ref):
    o_ref[...] = jnp.dot(x_ref[...], w_ref[...],
                         preferred_element_type=jnp.float32)


def _matmul(x, w):
    bm = 1000
    return pl.pallas_call(
        _mm_body,
        grid=(N_NODES // bm,),
        in_specs=[pl.BlockSpec((bm, D), lambda i: (i, 0)),
                  pl.BlockSpec((D, D), lambda i: (0, 0))],
        out_specs=pl.BlockSpec((bm, D), lambda i: (i, 0)),
        out_shape=jax.ShapeDtypeStruct((N_NODES, D), jnp.float32),
    )(x, w)


@functools.partial(
    pl.kernel,
    out_type=jax.ShapeDtypeStruct((NC, N_PAD, D), jnp.float32),
    mesh=plsc.VectorSubcoreMesh(core_axis_name="c", subcore_axis_name="s",
                                num_cores=NC, num_subcores=NS),
    scratch_types=[
        pltpu.VMEM((2 * BC, K), jnp.int32),
        pltpu.VMEM((K, D), jnp.float32),
        pltpu.VMEM((K, D), jnp.float32),
        pltpu.VMEM_SHARED((N_PAD, D), jnp.float32),
        pltpu.SemaphoreType.DMA,
        pltpu.SemaphoreType.DMA,
        pltpu.SemaphoreType.DMA,
        pltpu.SemaphoreType.DMA,
    ],
)
def _sc_scatter(mat_hbm, eidx_hbm, zero_hbm, out_hbm,
                idx_v, rows_a, rows_b, acc,
                gsem_a, gsem_b, ssem_a, ssem_b):
    cid = lax.axis_index("c")
    sid = lax.axis_index("s")
    wid = sid * NC + cid

    r0 = sid * ROWS_PER_TILE
    pltpu.sync_copy(zero_hbm.at[pl.ds(r0, ROWS_PER_TILE)],
                    acc.at[pl.ds(r0, ROWS_PER_TILE)])
    plsc.subcore_barrier()

    def blk_body(blk, carry):
        pltpu.sync_copy(eidx_hbm.at[wid, blk], idx_v)

        def pair(t, c2):
            ja = 2 * t
            jb = ja + 1
            pltpu.async_copy(mat_hbm.at[idx_v.at[ja]], rows_a, gsem_a).wait()
            pltpu.sync_copy(rows_a, acc.at[idx_v.at[BC + ja]], add=True)
            pltpu.async_copy(mat_hbm.at[idx_v.at[jb]], rows_b, gsem_b).wait()
            pltpu.sync_copy(rows_b, acc.at[idx_v.at[BC + jb]], add=True)
            return c2

        lax.fori_loop(0, BPAIRS, pair, 0)
        return carry

    lax.fori_loop(0, NBLK, blk_body, 0)

    plsc.subcore_barrier()
    pltpu.sync_copy(acc.at[pl.ds(r0, ROWS_PER_TILE)],
                    out_hbm.at[cid, pl.ds(r0, ROWS_PER_TILE)])


def _comb_body(p_ref, b_ref, o_ref):
    o_ref[...] = p_ref[0] + p_ref[1] + b_ref[...]


def _combine(p, b2):
    bm = 1000
    return pl.pallas_call(
        _comb_body,
        grid=(N_NODES // bm,),
        in_specs=[pl.BlockSpec((NC, bm, D), lambda i: (0, i, 0)),
                  pl.BlockSpec((1, D), lambda i: (0, 0))],
        out_specs=pl.BlockSpec((bm, D), lambda i: (i, 0)),
        out_shape=jax.ShapeDtypeStruct((N_NODES, D), jnp.float32),
    )(p, b2)


def kernel(input, edge_index, W, b):
    mat = _matmul(input, W)
    npad_e = E_PAD - N_EDGES
    src = jnp.concatenate(
        [edge_index[0], jnp.zeros((npad_e,), jnp.int32)]
    ).reshape(NW, NBLK, BC, K)
    dst = jnp.concatenate(
        [edge_index[1], jnp.full((npad_e,), N_PAD - 1, jnp.int32)]
    ).reshape(NW, NBLK, BC, K)
    eidx = jnp.concatenate([src, dst], axis=2)
    zeros = jnp.zeros((N_PAD, D), jnp.float32)
    partials = _sc_scatter(mat, eidx, zeros)
    return _combine(partials, b.reshape(1, D))


# speedup vs baseline: 1.0232x; 1.0232x over previous
import functools

import jax
import jax.numpy as jnp
from jax import lax
from jax.experimental import pallas as pl
from jax.experimental.pallas import tpu as pltpu
from jax.experimental.pallas import tpu_sc as plsc

N_NODES = 10000
D = 128
N_EDGES = 320000

NC = 2
NS = 16
NW = NC * NS
K = 80
NCHUNK = 128
NBLK = 2
BC = NCHUNK // NBLK
BPAIRS = BC // 2
EPW = NCHUNK * K
E_PAD = NW * EPW
ROWS_PER_TILE = 632
N_PAD = ROWS_PER_TILE * NS


def _mm_body(x_ref, w_ref, o_ref):
    o_ref[...] = jnp.dot(x_ref[...], w_ref[...],
                         preferred_element_type=jnp.float32)


def _matmul(x, w):
    bm = 1000
    return pl.pallas_call(
        _mm_body,
        grid=(N_NODES // bm,),
        in_specs=[pl.BlockSpec((bm, D), lambda i: (i, 0)),
                  pl.BlockSpec((D, D), lambda i: (0, 0))],
        out_specs=pl.BlockSpec((bm, D), lambda i: (i, 0)),
        out_shape=jax.ShapeDtypeStruct((N_NODES, D), jnp.float32),
    )(x, w)


@functools.partial(
    pl.kernel,
    out_type=jax.ShapeDtypeStruct((NC, N_PAD, D), jnp.float32),
    mesh=plsc.VectorSubcoreMesh(core_axis_name="c", subcore_axis_name="s",
                                num_cores=NC, num_subcores=NS),
    scratch_types=[
        pltpu.VMEM((BC, K), jnp.int32),
        pltpu.VMEM((BC, K), jnp.int32),
        pltpu.VMEM((K, D), jnp.float32),
        pltpu.VMEM((K, D), jnp.float32),
        pltpu.VMEM_SHARED((N_PAD, D), jnp.float32),
        pltpu.SemaphoreType.DMA,
        pltpu.SemaphoreType.DMA,
        pltpu.SemaphoreType.DMA,
        pltpu.SemaphoreType.DMA,
    ],
)
def _sc_scatter(mat_hbm, src_hbm, dst_hbm, zero_hbm, out_hbm,
                src_v, dst_v, rows_a, rows_b, acc,
                gsem_a, gsem_b, ssem_a, ssem_b):
    cid = lax.axis_index("c")
    sid = lax.axis_index("s")
    wid = sid * NC + cid

    r0 = sid * ROWS_PER_TILE
    pltpu.sync_copy(zero_hbm.at[pl.ds(r0, ROWS_PER_TILE)],
                    acc.at[pl.ds(r0, ROWS_PER_TILE)])
    plsc.subcore_barrier()

    def blk_body(blk, carry):
        pltpu.sync_copy(src_hbm.at[wid, blk], src_v)
        pltpu.sync_copy(dst_hbm.at[wid, blk], dst_v)

        def pair(t, c2):
            ja = 2 * t
            jb = ja + 1
            ga = pltpu.async_copy(mat_hbm.at[src_v.at[ja]], rows_a, gsem_a)
            gb = pltpu.async_copy(mat_hbm.at[src_v.at[jb]], rows_b, gsem_b)
            ga.wait()
            pltpu.sync_copy(rows_a, acc.at[dst_v.at[ja]], add=True)
            gb.wait()
            pltpu.sync_copy(rows_b, acc.at[dst_v.at[jb]], add=True)
            return c2

        lax.fori_loop(0, BPAIRS, pair, 0)
        return carry

    lax.fori_loop(0, NBLK, blk_body, 0)

    plsc.subcore_barrier()
    pltpu.sync_copy(acc.at[pl.ds(r0, ROWS_PER_TILE)],
                    out_hbm.at[cid, pl.ds(r0, ROWS_PER_TILE)])


def _comb_body(p_ref, b_ref, o_ref):
    o_ref[...] = p_ref[0] + p_ref[1] + b_ref[...]


def _combine(p, b2):
    bm = 1000
    return pl.pallas_call(
        _comb_body,
        grid=(N_NODES // bm,),
        in_specs=[pl.BlockSpec((NC, bm, D), lambda i: (0, i, 0)),
                  pl.BlockSpec((1, D), lambda i: (0, 0))],
        out_specs=pl.BlockSpec((bm, D), lambda i: (i, 0)),
        out_shape=jax.ShapeDtypeStruct((N_NODES, D), jnp.float32),
    )(p, b2)


def kernel(input, edge_index, W, b):
    mat = _matmul(input, W)
    npad_e = E_PAD - N_EDGES
    src = jnp.concatenate(
        [edge_index[0], jnp.zeros((npad_e,), jnp.int32)]
    ).reshape(NW, NBLK, BC, K)
    dst = jnp.concatenate(
        [edge_index[1], jnp.full((npad_e,), N_PAD - 1, jnp.int32)]
    ).reshape(NW, NBLK, BC, K)
    zeros = jnp.zeros((N_PAD, D), jnp.float32)
    partials = _sc_scatter(mat, src, dst, zeros)
    return _combine(partials, b.reshape(1, D))


# speedup vs baseline: 2.2964x; 2.2444x over previous
import functools

import jax
import jax.numpy as jnp
from jax import lax
from jax.experimental import pallas as pl
from jax.experimental.pallas import tpu as pltpu
from jax.experimental.pallas import tpu_sc as plsc

N_NODES = 10000
D = 128
N_EDGES = 320000

NC = 2
NS = 16
NW = NC * NS
EPW = N_EDGES // NW
K = 80
NCHUNK = EPW // K
ROWS_PER_TILE = 632
N_PAD = ROWS_PER_TILE * NS


def _mm_body(x_ref, w_ref, o_ref):
    o_ref[...] = jnp.dot(x_ref[...], w_ref[...],
                         preferred_element_type=jnp.float32)


def _matmul(x, w):
    bm = 1000
    return pl.pallas_call(
        _mm_body,
        grid=(N_NODES // bm,),
        in_specs=[pl.BlockSpec((bm, D), lambda i: (i, 0)),
                  pl.BlockSpec((D, D), lambda i: (0, 0))],
        out_specs=pl.BlockSpec((bm, D), lambda i: (i, 0)),
        out_shape=jax.ShapeDtypeStruct((N_NODES, D), jnp.float32),
    )(x, w)


@functools.partial(
    pl.kernel,
    out_type=jax.ShapeDtypeStruct((NC, N_PAD, D), jnp.float32),
    mesh=plsc.VectorSubcoreMesh(core_axis_name="c", subcore_axis_name="s",
                                num_cores=NC, num_subcores=NS),
    scratch_types=[
        pltpu.VMEM((NCHUNK, K), jnp.int32),
        pltpu.VMEM((NCHUNK, K), jnp.int32),
        pltpu.VMEM((K, D), jnp.float32),
        pltpu.VMEM_SHARED((N_PAD, D), jnp.float32),
        pltpu.SemaphoreType.DMA,
    ],
)
def _sc_scatter(mat_hbm, src_hbm, dst_hbm, zero_hbm, out_hbm,
                src_v, dst_v, rows_v, acc, sem):
    cid = lax.axis_index("c")
    sid = lax.axis_index("s")
    wid = sid * NC + cid

    pltpu.sync_copy(src_hbm.at[wid], src_v)
    pltpu.sync_copy(dst_hbm.at[wid], dst_v)
    r0 = sid * ROWS_PER_TILE
    pltpu.sync_copy(zero_hbm.at[pl.ds(r0, ROWS_PER_TILE)],
                    acc.at[pl.ds(r0, ROWS_PER_TILE)])
    plsc.subcore_barrier()

    def body(j, carry):
        pltpu.async_copy(mat_hbm.at[src_v.at[j]], rows_v, sem).wait()
        pltpu.sync_copy(rows_v, acc.at[dst_v.at[j]], add=True)
        return carry

    lax.fori_loop(0, NCHUNK, body, 0)

    plsc.subcore_barrier()
    pltpu.sync_copy(acc.at[pl.ds(r0, ROWS_PER_TILE)],
                    out_hbm.at[cid, pl.ds(r0, ROWS_PER_TILE)])


def _comb_body(p_ref, b_ref, o_ref):
    o_ref[...] = p_ref[0] + p_ref[1] + b_ref[...]


def _combine(p, b2):
    bm = 1000
    return pl.pallas_call(
        _comb_body,
        grid=(N_NODES // bm,),
        in_specs=[pl.BlockSpec((NC, bm, D), lambda i: (0, i, 0)),
                  pl.BlockSpec((1, D), lambda i: (0, 0))],
        out_specs=pl.BlockSpec((bm, D), lambda i: (i, 0)),
        out_shape=jax.ShapeDtypeStruct((N_NODES, D), jnp.float32),
    )(p, b2)


def kernel(input, edge_index, W, b):
    mat = _matmul(input, W)
    src = edge_index[0].reshape(NW, NCHUNK, K)
    dst = edge_index[1].reshape(NW, NCHUNK, K)
    zeros = jnp.zeros((N_PAD, D), jnp.float32)
    partials = _sc_scatter(mat, src, dst, zeros)
    return _combine(partials, b.reshape(1, D))
